# baseline (speedup 1.0000x reference)
import functools

import jax
import jax.numpy as jnp
from jax import lax
from jax.experimental import pallas as pl
from jax.experimental.pallas import tpu as pltpu
from jax.experimental.pallas import tpu_sc as plsc

NC = 2
NS = 16
NW = NC * NS

VOCAB = 1000
D = 64
B_TOTAL = 1024 * 200
CHUNK = 128
PACK = CHUNK // 2
N_CHUNKS = B_TOTAL // (NW * CHUNK)
PER_W = N_CHUNKS * CHUNK
RING = 2
L = 16


def _body(idx_hbm, table_hbm, out_hbm, idx_v, table_v, w, wsem):
    wid = lax.axis_index("s") * NC + lax.axis_index("c")
    pbase = wid * (PER_W // 2)

    pltpu.sync_copy(idx_hbm.at[wid], idx_v)
    pltpu.sync_copy(table_hbm, table_v)

    lanes = lax.iota(jnp.int32, L)
    half = lax.rem(lanes, 2) * D
    pair = lax.div(lanes, 2)

    @pl.loop(0, N_CHUNKS)
    def _(c):
        slot = lax.rem(c, RING)

        @pl.when(c >= RING)
        def _():
            pltpu.make_async_copy(
                w.at[slot],
                out_hbm.at[pl.ds(pbase + (c - RING) * PACK, PACK)],
                wsem.at[slot]).wait()

        @pl.loop(0, CHUNK // L)
        def _(rg):
            rids = idx_v[c, pl.ds(rg * L, L)]
            addr0 = rids * D
            prow = pair + rg * (L // 2)
            for word in range(D):
                val = plsc.load_gather(table_v, [addr0 + word])
                plsc.store_scatter(w.at[slot], [prow, half + word], val)

        pltpu.async_copy(
            w.at[slot], out_hbm.at[pl.ds(pbase + c * PACK, PACK)],
            wsem.at[slot])

    for s in range(RING):
        c = N_CHUNKS - RING + s
        pltpu.make_async_copy(
            w.at[lax.rem(jnp.int32(c), RING)],
            out_hbm.at[pl.ds(pbase + c * PACK, PACK)],
            wsem.at[lax.rem(jnp.int32(c), RING)]).wait()


@jax.jit
def _lookup(idx, table_flat):
    mesh = plsc.VectorSubcoreMesh(core_axis_name="c", subcore_axis_name="s")
    f = pl.kernel(
        _body,
        out_type=jax.ShapeDtypeStruct((B_TOTAL // 2, 2 * D), jnp.float32),
        mesh=mesh,
        scratch_types=[
            pltpu.VMEM((N_CHUNKS, CHUNK), jnp.int32),
            pltpu.VMEM((VOCAB * D,), jnp.float32),
            pltpu.VMEM((RING, PACK, 2 * D), jnp.float32),
            pltpu.SemaphoreType.DMA((RING,)),
        ],
        compiler_params=pltpu.CompilerParams(needs_layout_passes=False),
    )
    return f(idx, table_flat)


def kernel(indices, in_embed, out_embed):
    del out_embed
    bsz, hist = indices.shape
    idx = indices.astype(jnp.int32).reshape(NW, N_CHUNKS, CHUNK)
    out = _lookup(idx, in_embed.reshape(-1))
    return out.reshape(bsz, hist, D)


# speedup vs baseline: 2.5834x; 2.5834x over previous
import functools

import jax
import jax.numpy as jnp
from jax import lax
from jax.experimental import pallas as pl
from jax.experimental.pallas import tpu as pltpu
from jax.experimental.pallas import tpu_sc as plsc

NC = 2
NS = 16
NW = NC * NS

VOCAB = 1000
D = 64
DP = 128
B_TOTAL = 1024 * 200
CHUNK = 128
PACK = CHUNK // 2
N_CHUNKS = B_TOTAL // (NW * CHUNK)
PER_W = N_CHUNKS * CHUNK
RING = 2
L = 16


def _body(idx_hbm, table_hbm, out_hbm, idx_v, g, w, gsem, wsem):
    wid = lax.axis_index("s") * NC + lax.axis_index("c")
    pbase = wid * (PER_W // 2)

    pltpu.sync_copy(idx_hbm.at[wid], idx_v)

    def gather(c, slot):
        pltpu.async_copy(
            table_hbm.at[idx_v.at[c]], g.at[slot], gsem.at[slot])

    def wait_gather(c, slot):
        pltpu.make_async_copy(
            table_hbm.at[idx_v.at[c]], g.at[slot], gsem.at[slot]).wait()

    def write(c, slot):
        pltpu.async_copy(
            w.at[slot], out_hbm.at[pl.ds(pbase + c * PACK, PACK)],
            wsem.at[slot])

    def wait_write(c, slot):
        pltpu.make_async_copy(
            w.at[slot], out_hbm.at[pl.ds(pbase + c * PACK, PACK)],
            wsem.at[slot]).wait()

    for j in range(RING):
        gather(j, j)

    @pl.loop(0, N_CHUNKS)
    def _(c):
        slot = lax.rem(c, RING)

        wait_gather(c, slot)

        @pl.when(c >= RING)
        def _():
            wait_write(c - RING, slot)

        gs = g.at[slot]
        ws = w.at[slot]

        @pl.loop(0, PACK // (L // 2))
        def _(pg):
            for pp in range(L // 2):
                p = pg * (L // 2) + pp
                for cg in range(D // L):
                    lo = gs[2 * p, pl.ds(cg * L, L)]
                    hi = gs[2 * p + 1, pl.ds(cg * L, L)]
                    ws[p, pl.ds(cg * L, L)] = lo
                    ws[p, pl.ds(D + cg * L, L)] = hi

        @pl.when(c + RING < N_CHUNKS)
        def _():
            gather(c + RING, slot)

        write(c, slot)

    for s in range(RING):
        c = N_CHUNKS - RING + s
        wait_write(c, c % RING)


@jax.jit
def _lookup(idx, table_padded):
    mesh = plsc.VectorSubcoreMesh(core_axis_name="c", subcore_axis_name="s")
    f = pl.kernel(
        _body,
        out_type=jax.ShapeDtypeStruct((B_TOTAL // 2, 2 * D), jnp.float32),
        mesh=mesh,
        scratch_types=[
            pltpu.VMEM((N_CHUNKS, CHUNK), jnp.int32),
            pltpu.VMEM((RING, CHUNK, DP), jnp.float32),
            pltpu.VMEM((RING, PACK, 2 * D), jnp.float32),
            pltpu.SemaphoreType.DMA((RING,)),
            pltpu.SemaphoreType.DMA((RING,)),
        ],
    )
    return f(idx, table_padded)


def kernel(indices, in_embed, out_embed):
    del out_embed
    bsz, hist = indices.shape
    idx = indices.astype(jnp.int32).reshape(NW, N_CHUNKS, CHUNK)
    table_padded = jnp.pad(in_embed, ((0, 0), (0, DP - D)))
    out = _lookup(idx, table_padded)
    return out.reshape(bsz, hist, D)


# speedup vs baseline: 3.6919x; 1.4291x over previous
import functools

import jax
import jax.numpy as jnp
from jax import lax
from jax.experimental import pallas as pl
from jax.experimental.pallas import tpu as pltpu
from jax.experimental.pallas import tpu_sc as plsc

NC = 2
NS = 16
NW = NC * NS

VOCAB = 1000
D = 64
B_TOTAL = 1024 * 200
CHUNK = 128
N_CHUNKS = B_TOTAL // (NW * CHUNK)
PER_W = N_CHUNKS * CHUNK
RING = 10
LOOKAHEAD = 5


def _body(idx_hbm, table_hbm, out_hbm, idx_v, tsh, rows, gsem, wsem):
    wid = lax.axis_index("s") * NC + lax.axis_index("c")
    base = wid * PER_W

    @pl.when(lax.axis_index("s") == 0)
    def _():
        pltpu.sync_copy(table_hbm, tsh)

    pltpu.sync_copy(idx_hbm.at[wid], idx_v)
    plsc.subcore_barrier()

    def gather(c, slot):
        pltpu.async_copy(tsh.at[idx_v.at[c]], rows.at[slot], gsem.at[slot])

    def wait_gather(c, slot):
        pltpu.make_async_copy(
            tsh.at[idx_v.at[c]], rows.at[slot], gsem.at[slot]).wait()

    def write(c, slot):
        pltpu.async_copy(
            rows.at[slot], out_hbm.at[pl.ds(base + c * CHUNK, CHUNK)],
            wsem.at[slot])

    def wait_write(c, slot):
        pltpu.make_async_copy(
            rows.at[slot], out_hbm.at[pl.ds(base + c * CHUNK, CHUNK)],
            wsem.at[slot]).wait()

    for j in range(LOOKAHEAD):
        gather(j, j)

    @pl.loop(0, N_CHUNKS, step=RING)
    def _(g):
        for j in range(RING):
            c = g + j
            bf = (j + LOOKAHEAD) % RING

            def reuse(c=c, bf=bf):
                wait_write(c - LOOKAHEAD, bf)
                gather(c + LOOKAHEAD, bf)

            if j < LOOKAHEAD:
                @pl.when(g > 0)
                def _(reuse=reuse):
                    reuse()

                @pl.when(g == 0)
                def _(c=c, bf=bf):
                    gather(c + LOOKAHEAD, bf)
            else:
                @pl.when(g < N_CHUNKS - RING)
                def _(reuse=reuse):
                    reuse()

                @pl.when(g == N_CHUNKS - RING)
                def _(c=c, bf=bf):
                    wait_write(c - LOOKAHEAD, bf)

            wait_gather(c, j)
            write(c, j)

    for j in range(RING - LOOKAHEAD, RING):
        wait_write(N_CHUNKS - RING + j, j)


@jax.jit
def _lookup(idx, in_embed):
    mesh = plsc.VectorSubcoreMesh(core_axis_name="c", subcore_axis_name="s")
    f = pl.kernel(
        _body,
        out_type=jax.ShapeDtypeStruct((B_TOTAL, D), jnp.float32),
        mesh=mesh,
        scratch_types=[
            pltpu.VMEM((N_CHUNKS, CHUNK), jnp.int32),
            pltpu.VMEM_SHARED((VOCAB, D), jnp.float32),
            pltpu.VMEM((RING, CHUNK, D), jnp.float32),
            pltpu.SemaphoreType.DMA((RING,)),
            pltpu.SemaphoreType.DMA((RING,)),
        ],
        compiler_params=pltpu.CompilerParams(use_tc_tiling_on_sc=False),
    )
    return f(idx, in_embed)


def kernel(indices, in_embed, out_embed):
    del out_embed
    bsz, hist = indices.shape
    idx = indices.astype(jnp.int32).reshape(NW, N_CHUNKS, CHUNK)
    out = _lookup(idx, in_embed)
    return out.reshape(bsz, hist, D)
